# baseline (speedup 1.0000x reference)
import jax
import jax.numpy as jnp
from jax.experimental import pallas as pl

N_CB = 8
K = 1024
D = 64
CC = 0.5
PROJ = 192
W2V = 768
B = 16
T = 2048
N = B * T
BLK = 512
G = N // BLK
_SQRT_HALF = 0.7071067811865476


def _vq_core(a, x, e, c):
    mm = jax.lax.dot_general(
        x, e, (((1,), (1,)), ((), ())), preferred_element_type=jnp.float32)
    dist = a - 2.0 * mm + c
    m = jnp.min(dist, axis=1, keepdims=True)
    iota = jax.lax.broadcasted_iota(jnp.int32, (BLK, K), 1)
    idxk = jnp.min(jnp.where(dist == m, iota, K), axis=1, keepdims=True)
    oh = (iota == idxk).astype(jnp.float32)
    zq = jax.lax.dot_general(
        oh, e, (((1,), (0,)), ((), ())),
        preferred_element_type=jnp.float32,
        precision=jax.lax.Precision.HIGHEST)
    t = zq - x
    s = x + t
    r = x - s
    return idxk, s, r, t


def _stage_kernel(a_ref, x_ref, e_ref, c_ref,
                  idx_ref, s_ref, r_ref, vq_ref):
    idxk, s, r, t = _vq_core(a_ref[...], x_ref[...], e_ref[...], c_ref[...])
    idx_ref[...] = idxk
    s_ref[...] = s
    r_ref[...] = r

    @pl.when(pl.program_id(0) == 0)
    def _():
        vq_ref[...] = jnp.zeros_like(vq_ref)

    vq_ref[...] += jnp.sum(t * t, keepdims=True)


def _stage0_kernel(a_ref, x_ref, e_ref, c_ref, w1_ref, b1_ref, w2_ref,
                   b2_ref, tgt_ref,
                   idx_ref, s_ref, r_ref, vq_ref, sem_ref):
    idxk, s, r, t = _vq_core(a_ref[...], x_ref[...], e_ref[...], c_ref[...])
    idx_ref[...] = idxk
    s_ref[...] = s
    r_ref[...] = r

    h = jax.lax.dot_general(
        s, w1_ref[...], (((1,), (0,)), ((), ())),
        preferred_element_type=jnp.float32) + b1_ref[...]
    g = 0.5 * h * (1.0 + jax.lax.erf(h * _SQRT_HALF))
    pred = jax.lax.dot_general(
        g, w2_ref[...], (((1,), (0,)), ((), ())),
        preferred_element_type=jnp.float32) + b2_ref[...]
    dd = pred - tgt_ref[...]

    @pl.when(pl.program_id(0) == 0)
    def _():
        vq_ref[...] = jnp.zeros_like(vq_ref)
        sem_ref[...] = jnp.zeros_like(sem_ref)

    vq_ref[...] += jnp.sum(t * t, keepdims=True)
    sem_ref[...] += jnp.sum(dd * dd, keepdims=True)


_ROW = lambda i: (i, 0)
_CONST = lambda i: (0, 0)


def _stage_call(a, x, e, c):
    return pl.pallas_call(
        _stage_kernel,
        grid=(G,),
        in_specs=[pl.BlockSpec((BLK, 1), _ROW),
                  pl.BlockSpec((BLK, D), _ROW),
                  pl.BlockSpec((K, D), _CONST),
                  pl.BlockSpec((1, K), _CONST)],
        out_specs=[pl.BlockSpec((BLK, 1), _ROW),
                   pl.BlockSpec((BLK, D), _ROW),
                   pl.BlockSpec((BLK, D), _ROW),
                   pl.BlockSpec((1, 1), _CONST)],
        out_shape=[jax.ShapeDtypeStruct((N, 1), jnp.int32),
                   jax.ShapeDtypeStruct((N, D), jnp.float32),
                   jax.ShapeDtypeStruct((N, D), jnp.float32),
                   jax.ShapeDtypeStruct((1, 1), jnp.float32)],
    )(a, x, e, c)


def _stage0_call(a, x, e, c, W1, b1, W2, b2, tgt):
    return pl.pallas_call(
        _stage0_kernel,
        grid=(G,),
        in_specs=[pl.BlockSpec((BLK, 1), _ROW),
                  pl.BlockSpec((BLK, D), _ROW),
                  pl.BlockSpec((K, D), _CONST),
                  pl.BlockSpec((1, K), _CONST),
                  pl.BlockSpec((D, PROJ), _CONST),
                  pl.BlockSpec((1, PROJ), _CONST),
                  pl.BlockSpec((PROJ, W2V), _CONST),
                  pl.BlockSpec((1, W2V), _CONST),
                  pl.BlockSpec((BLK, W2V), _ROW)],
        out_specs=[pl.BlockSpec((BLK, 1), _ROW),
                   pl.BlockSpec((BLK, D), _ROW),
                   pl.BlockSpec((BLK, D), _ROW),
                   pl.BlockSpec((1, 1), _CONST),
                   pl.BlockSpec((1, 1), _CONST)],
        out_shape=[jax.ShapeDtypeStruct((N, 1), jnp.int32),
                   jax.ShapeDtypeStruct((N, D), jnp.float32),
                   jax.ShapeDtypeStruct((N, D), jnp.float32),
                   jax.ShapeDtypeStruct((1, 1), jnp.float32),
                   jax.ShapeDtypeStruct((1, 1), jnp.float32)],
    )(a, x, e, c, W1, b1, W2, b2, tgt)


def kernel(z, w2v_targets, codebooks, W1, b1, W2, b2):
    resid = z.reshape(N, D)
    tgt = w2v_targets.reshape(N, W2V)
    b1r = b1[None, :]
    b2r = b2[None, :]

    idxs = []
    stes = []
    vq_parts = []
    sem_part = None
    for i in range(N_CB):
        a = jnp.sum(resid ** 2, axis=1, keepdims=True)
        c = jnp.sum(codebooks[i] ** 2, axis=1)[None, :]
        if i == 0:
            idx, s, resid, vq_p, sem_part = _stage0_call(
                a, resid, codebooks[i], c, W1, b1r, W2, b2r, tgt)
        else:
            idx, s, resid, vq_p = _stage_call(a, resid, codebooks[i], c)
        idxs.append(idx)
        stes.append(s)
        vq_parts.append(vq_p)

    z_q_total = stes[0]
    for s in stes[1:]:
        z_q_total = z_q_total + s
    z_q_total = z_q_total.reshape(B, T, D)

    all_codes = jnp.concatenate(idxs, axis=1).reshape(B, T, N_CB)

    vq_loss = jnp.float32(0.0)
    denom = jnp.float32(N * D)
    for vq_p in vq_parts:
        e_mean = vq_p[0, 0] / denom
        vq_loss = vq_loss + (e_mean + CC * e_mean)

    semantic_loss = sem_part[0, 0] / jnp.float32(N * W2V)

    return (z_q_total, all_codes, all_codes[..., 0], vq_loss, semantic_loss)


# speedup vs baseline: 2.8859x; 2.8859x over previous
import jax
import jax.numpy as jnp
from jax.experimental import pallas as pl

N_CB = 8
K = 1024
D = 64
CC = 0.5
PROJ = 192
W2V = 768
B = 16
T = 2048
N = B * T
BLK = 2048
G = N // BLK
_SQRT_HALF = 0.7071067811865476


def _colsum_sq_xla_order(x):
    xx = x * x
    t = xx[0:8, :]
    for v in range(1, 8):
        t = t + xx[8 * v:8 * v + 8, :]
    s = t[0:4, :] + t[4:8, :]
    s = s[0:2, :] + s[2:4, :]
    return s[0:1, :] + s[1:2, :]


def _vq_stage(x, e, est, c):
    a = _colsum_sq_xla_order(x)
    mm = jax.lax.dot_general(
        e, x, (((1,), (0,)), ((), ())), preferred_element_type=jnp.float32)
    dist = a - 2.0 * mm + c
    m = jnp.min(dist, axis=0, keepdims=True)
    iota = jax.lax.broadcasted_iota(jnp.int32, (K, 1), 0).astype(jnp.float32)
    sel = jnp.where(dist == m, iota, jnp.float32(K))
    idxf = jnp.min(sel, axis=0, keepdims=True)
    oh = (sel == idxf).astype(jnp.float32)
    zq3 = jax.lax.dot_general(
        est, oh, (((1,), (0,)), ((), ())),
        preferred_element_type=jnp.float32)
    zq = (zq3[0:D, :] + zq3[128:128 + D, :]) + zq3[256:256 + D, :]
    t = zq - x
    s = x + t
    r = x - s
    return idxf.astype(jnp.int32), s, r, t


def _stage_kernel(x_ref, e_ref, est_ref, c_ref,
                  idx_ref, s_ref, r_ref, vq_ref):
    idxk, s, r, t = _vq_stage(x_ref[...], e_ref[...], est_ref[...], c_ref[...])
    idx_ref[...] = idxk
    s_ref[...] = s
    r_ref[...] = r

    @pl.when(pl.program_id(0) == 0)
    def _():
        vq_ref[...] = jnp.zeros_like(vq_ref)

    vq_ref[...] += jnp.sum(t * t, keepdims=True)


def _stage0_kernel(x_ref, e_ref, est_ref, c_ref, w1_ref, b1_ref,
                   w2_ref, b2_ref, tgt_ref,
                   idx_ref, s_ref, r_ref, vq_ref, sem_ref):
    idxk, s, r, t = _vq_stage(x_ref[...], e_ref[...], est_ref[...], c_ref[...])
    idx_ref[...] = idxk
    s_ref[...] = s
    r_ref[...] = r

    srow = jnp.transpose(s, (1, 0))
    h = jax.lax.dot_general(
        srow, w1_ref[...], (((1,), (0,)), ((), ())),
        preferred_element_type=jnp.float32) + b1_ref[...]
    g = 0.5 * h * (1.0 + jax.lax.erf(h * _SQRT_HALF))
    pred = jax.lax.dot_general(
        g, w2_ref[...], (((1,), (0,)), ((), ())),
        preferred_element_type=jnp.float32) + b2_ref[...]
    dd = pred - tgt_ref[...]

    @pl.when(pl.program_id(0) == 0)
    def _():
        vq_ref[...] = jnp.zeros_like(vq_ref)
        sem_ref[...] = jnp.zeros_like(sem_ref)

    vq_ref[...] += jnp.sum(t * t, keepdims=True)
    sem_ref[...] += jnp.sum(dd * dd, keepdims=True)


_COL = lambda i: (0, i)
_ROW = lambda i: (i, 0)
_CONST = lambda i: (0, 0)


def _stage_call(x, e, est, c):
    return pl.pallas_call(
        _stage_kernel,
        grid=(G,),
        in_specs=[pl.BlockSpec((D, BLK), _COL),
                  pl.BlockSpec((K, D), _CONST),
                  pl.BlockSpec((384, K), _CONST),
                  pl.BlockSpec((K, 1), _CONST)],
        out_specs=[pl.BlockSpec((1, BLK), _COL),
                   pl.BlockSpec((D, BLK), _COL),
                   pl.BlockSpec((D, BLK), _COL),
                   pl.BlockSpec((1, 1), _CONST)],
        out_shape=[jax.ShapeDtypeStruct((1, N), jnp.int32),
                   jax.ShapeDtypeStruct((D, N), jnp.float32),
                   jax.ShapeDtypeStruct((D, N), jnp.float32),
                   jax.ShapeDtypeStruct((1, 1), jnp.float32)],
    )(x, e, est, c)


def _stage0_call(x, e, est, c, W1, b1, W2, b2, tgt):
    return pl.pallas_call(
        _stage0_kernel,
        grid=(G,),
        in_specs=[pl.BlockSpec((D, BLK), _COL),
                  pl.BlockSpec((K, D), _CONST),
                  pl.BlockSpec((384, K), _CONST),
                  pl.BlockSpec((K, 1), _CONST),
                  pl.BlockSpec((D, PROJ), _CONST),
                  pl.BlockSpec((1, PROJ), _CONST),
                  pl.BlockSpec((PROJ, W2V), _CONST),
                  pl.BlockSpec((1, W2V), _CONST),
                  pl.BlockSpec((BLK, W2V), _ROW)],
        out_specs=[pl.BlockSpec((1, BLK), _COL),
                   pl.BlockSpec((D, BLK), _COL),
                   pl.BlockSpec((D, BLK), _COL),
                   pl.BlockSpec((1, 1), _CONST),
                   pl.BlockSpec((1, 1), _CONST)],
        out_shape=[jax.ShapeDtypeStruct((1, N), jnp.int32),
                   jax.ShapeDtypeStruct((D, N), jnp.float32),
                   jax.ShapeDtypeStruct((D, N), jnp.float32),
                   jax.ShapeDtypeStruct((1, 1), jnp.float32),
                   jax.ShapeDtypeStruct((1, 1), jnp.float32)],
    )(x, e, est, c, W1, b1, W2, b2, tgt)


def _split3_t(e):
    mask = jnp.uint32(0xFFFF0000)
    u = jax.lax.bitcast_convert_type(e, jnp.uint32)
    hi = jax.lax.bitcast_convert_type(u & mask, jnp.float32)
    r1 = e - hi
    u1 = jax.lax.bitcast_convert_type(r1, jnp.uint32)
    mid = jax.lax.bitcast_convert_type(u1 & mask, jnp.float32)
    lo = r1 - mid
    z64 = jnp.zeros_like(e)
    return jnp.concatenate(
        [hi.T, z64.T, mid.T, z64.T, lo.T, z64.T], axis=0)


def kernel(z, w2v_targets, codebooks, W1, b1, W2, b2):
    xT = z.reshape(N, D).T
    tgt = w2v_targets.reshape(N, W2V)
    b1r = b1[None, :]
    b2r = b2[None, :]

    idxs = []
    stes = []
    vq_parts = []
    sem_part = None
    resid = xT
    for i in range(N_CB):
        e = codebooks[i]
        c = jnp.sum(e ** 2, axis=1)[:, None]
        est = _split3_t(e)
        if i == 0:
            idx, s, resid, vq_p, sem_part = _stage0_call(
                resid, e, est, c, W1, b1r, W2, b2r, tgt)
        else:
            idx, s, resid, vq_p = _stage_call(resid, e, est, c)
        idxs.append(idx)
        stes.append(s)
        vq_parts.append(vq_p)

    zqtot = stes[0]
    for s in stes[1:]:
        zqtot = zqtot + s
    z_q_total = zqtot.T.reshape(B, T, D)

    all_codes = jnp.concatenate(idxs, axis=0).T.reshape(B, T, N_CB)

    vq_loss = jnp.float32(0.0)
    denom = jnp.float32(N * D)
    for vq_p in vq_parts:
        e_mean = vq_p[0, 0] / denom
        vq_loss = vq_loss + (e_mean + CC * e_mean)

    semantic_loss = sem_part[0, 0] / jnp.float32(N * W2V)

    return (z_q_total, all_codes, all_codes[..., 0], vq_loss, semantic_loss)
